# baseline (speedup 1.0000x reference)
import jax
import jax.numpy as jnp
from jax.experimental import pallas as pl
from jax.experimental.pallas import tpu as pltpu

DEPTH_ = 100000
BATCH_ = 1024

BLOCK_C = 1024


def _onehot_block(idx_ref, col_ref, out_ref):
    idx = idx_ref[...]
    col = col_ref[...]
    out_ref[...] = (col == idx).astype(jnp.float32)


def kernel(input):
    idx = input.astype(jnp.int32)
    col = jax.lax.broadcasted_iota(jnp.int32, (1, DEPTH_), 1)
    grid = (pl.cdiv(DEPTH_, BLOCK_C),)
    out = pl.pallas_call(
        _onehot_block,
        grid=grid,
        in_specs=[
            pl.BlockSpec((BATCH_, 1), lambda j: (0, 0)),
            pl.BlockSpec((1, BLOCK_C), lambda j: (0, j)),
        ],
        out_specs=pl.BlockSpec((BATCH_, BLOCK_C), lambda j: (0, j)),
        out_shape=jax.ShapeDtypeStruct((BATCH_, DEPTH_), jnp.float32),
    )(idx, col)
    return out


# speedup vs baseline: 1.0002x; 1.0002x over previous
import jax
import jax.numpy as jnp
from jax.experimental import pallas as pl
from jax.experimental.pallas import tpu as pltpu

DEPTH_ = 100000
BATCH_ = 1024

BLOCK_C = 1024


def _onehot_block(idx_ref, col_ref, out_ref):
    idx = idx_ref[...]
    col = col_ref[...]
    out_ref[...] = (col == idx).astype(jnp.float32)


def kernel(input):
    idx = input.astype(jnp.int32)
    col = jax.lax.broadcasted_iota(jnp.int32, (1, DEPTH_), 1)
    grid = (pl.cdiv(DEPTH_, BLOCK_C),)
    out = pl.pallas_call(
        _onehot_block,
        grid=grid,
        in_specs=[
            pl.BlockSpec((BATCH_, 1), lambda j: (0, 0)),
            pl.BlockSpec((1, BLOCK_C), lambda j: (0, j)),
        ],
        out_specs=pl.BlockSpec((BATCH_, BLOCK_C), lambda j: (0, j)),
        out_shape=jax.ShapeDtypeStruct((BATCH_, DEPTH_), jnp.float32),
        compiler_params=pltpu.CompilerParams(
            dimension_semantics=("parallel",)
        ),
    )(idx, col)
    return out
